# Initial kernel scaffold; baseline (speedup 1.0000x reference)
import jax
import jax.numpy as jnp
from jax.experimental import pallas as pl


def kernel(x_node, edge_index_e0, edge_index_e1, Wp, bp, W0_e0, b0_e0, W0_e1, b0_e1, W1_e0, b1_e0, W1_e1, b1_e1, Wlin, blin):
    raise NotImplementedError("write your pallas kernel here")


# speedup vs baseline: 3.2191x; 3.2191x over previous
import functools

import jax
import jax.numpy as jnp
from jax import lax
from jax.experimental import pallas as pl
from jax.experimental.pallas import tpu as pltpu
from jax.experimental.pallas import tpu_sc as plsc

N = 100000
E = 600000
D = 128
H = 128
O = 64

L = 16
NC = 2
NS = 16
NSL = H // L
SL_PER_CORE = NSL // NC

CHUNK = 128
KI = 4
GROUPS = 74
HALF = GROUPS // 2
ROWS_PER_TILE = GROUPS * KI
EPAD = ROWS_PER_TILE * NS * CHUNK
EROWS = EPAD // CHUNK

NPAD = 100096
ACC_ROWS_PER_TILE = NPAD // NS
WB = 256
WB_FULL = ACC_ROWS_PER_TILE // WB
WB_TAIL = ACC_ROWS_PER_TILE - WB_FULL * WB

_mesh = plsc.VectorSubcoreMesh(core_axis_name="c", subcore_axis_name="s")
_sc_params = pltpu.CompilerParams(use_tc_tiling_on_sc=False)


def _fill_const(buf, rows, value):
    def body(i, _):
        buf[i] = jnp.full((L,), value, jnp.float32)
        return _
    lax.fori_loop(0, rows, body, None)


def _zero_acc_rows(acc, zbuf, r0):
    def body(i, _):
        pltpu.sync_copy(zbuf, acc.at[pl.ds(r0 + i * WB, WB)])
        return _
    lax.fori_loop(0, WB_FULL, body, None)
    pltpu.sync_copy(zbuf.at[pl.ds(0, WB_TAIL)],
                    acc.at[pl.ds(r0 + WB_FULL * WB, WB_TAIL)])


def _count_body(dst0_hbm, dst1_hbm, inv0_hbm, inv1_hbm,
                acc, ones, dbuf, wbuf, zbuf, csem):
    cid = lax.axis_index("c")
    sid = lax.axis_index("s")
    r0 = sid * ACC_ROWS_PER_TILE

    _fill_const(zbuf, WB, 0.0)
    _fill_const(ones, CHUNK, 1.0)
    _zero_acc_rows(acc, zbuf, r0)
    plsc.subcore_barrier()

    def scatter_counts(dst_hbm):
        ebase = sid * ROWS_PER_TILE

        def group(g, carry):
            pltpu.sync_copy(dst_hbm.at[pl.ds(ebase + g * KI, KI)], dbuf)
            dd = [pltpu.async_copy(ones, acc.at[dbuf.at[j]], csem, add=True)
                  for j in range(KI)]
            for d in dd:
                d.wait()
            return carry
        lax.fori_loop(0, GROUPS, group, None)

    def writeback(inv_hbm):
        def inv_chunk(off, sz):
            pltpu.sync_copy(acc.at[pl.ds(off, sz)], wbuf.at[pl.ds(0, sz)])

            def body(i, _):
                c = wbuf[i]
                wbuf[i] = 1.0 / jnp.maximum(c, 1.0)
                return _
            lax.fori_loop(0, sz, body, None)
            pltpu.sync_copy(wbuf.at[pl.ds(0, sz)], inv_hbm.at[pl.ds(off, sz)])

        def chunk(k, _):
            inv_chunk(r0 + k * WB, WB)
            return _
        lax.fori_loop(0, WB_FULL, chunk, None)
        inv_chunk(r0 + WB_FULL * WB, WB_TAIL)

    @pl.when(cid == 0)
    def _():
        scatter_counts(dst0_hbm)
        plsc.subcore_barrier()
        writeback(inv0_hbm)

    @pl.when(cid == 1)
    def _():
        scatter_counts(dst1_hbm)
        plsc.subcore_barrier()
        writeback(inv1_hbm)


_count_call = pl.kernel(
    _count_body,
    out_type=(jax.ShapeDtypeStruct((NPAD, L), jnp.float32),
              jax.ShapeDtypeStruct((NPAD, L), jnp.float32)),
    mesh=_mesh,
    compiler_params=_sc_params,
    scratch_types=[
        pltpu.VMEM_SHARED((NPAD, L), jnp.float32),
        pltpu.VMEM((CHUNK, L), jnp.float32),
        pltpu.VMEM((KI, CHUNK), jnp.int32),
        pltpu.VMEM((WB, L), jnp.float32),
        pltpu.VMEM((WB, L), jnp.float32),
        pltpu.SemaphoreType.DMA,
    ],
)


def _segsum_body(tab_hbm, src3d_hbm, dst2d_hbm, out_hbm,
                 acc, sbuf, dbuf, rows, zbuf, gsem0, gsem1, ssem0, ssem1):
    cid = lax.axis_index("c")
    sid = lax.axis_index("s")
    r0 = sid * ACC_ROWS_PER_TILE
    ebase = sid * ROWS_PER_TILE

    _fill_const(zbuf, WB, 0.0)
    _zero_acc_rows(acc, zbuf, r0)
    plsc.subcore_barrier()

    for fs in range(NSL):
        @pl.when(cid == fs // SL_PER_CORE)
        def _():
            gsems = (gsem0, gsem1)
            ssems = (ssem0, ssem1)

            def idx_load(b, g):
                pltpu.sync_copy(src3d_hbm.at[fs, pl.ds(ebase + g * KI, KI)],
                                sbuf.at[b])
                pltpu.sync_copy(dst2d_hbm.at[pl.ds(ebase + g * KI, KI)],
                                dbuf.at[b])

            def fire_gather(b):
                for j in range(KI):
                    pltpu.async_copy(tab_hbm.at[sbuf.at[b, j]],
                                     rows.at[b, j], gsems[b])

            def drain_gather(b):
                for j in range(KI):
                    pltpu.make_async_copy(tab_hbm.at[sbuf.at[b, j]],
                                          rows.at[b, j], gsems[b]).wait()

            def fire_scatter(b):
                for j in range(KI):
                    pltpu.async_copy(rows.at[b, j], acc.at[dbuf.at[b, j]],
                                     ssems[b], add=True)

            def drain_scatter(b):
                for j in range(KI):
                    pltpu.make_async_copy(rows.at[b, j],
                                          acc.at[dbuf.at[b, j]],
                                          ssems[b]).wait()

            for b in range(2):
                idx_load(b, b)
                fire_gather(b)

            def pairstep(p, carry):
                for b in range(2):
                    g = 2 * p + b
                    drain_gather(b)
                    fire_scatter(b)
                    drain_scatter(b)

                    @pl.when(g + 2 < GROUPS)
                    def _():
                        idx_load(b, g + 2)
                        fire_gather(b)
                return carry
            lax.fori_loop(0, HALF, pairstep, None)
            plsc.subcore_barrier()

            def wb_chunk(off, sz):
                pltpu.sync_copy(acc.at[pl.ds(off, sz)],
                                out_hbm.at[fs, pl.ds(off, sz)])
                pltpu.sync_copy(zbuf.at[pl.ds(0, sz)],
                                acc.at[pl.ds(off, sz)])

            def chunk(k, _):
                wb_chunk(r0 + k * WB, WB)
                return _
            lax.fori_loop(0, WB_FULL, chunk, None)
            wb_chunk(r0 + WB_FULL * WB, WB_TAIL)
            plsc.subcore_barrier()


_segsum_call = pl.kernel(
    _segsum_body,
    out_type=jax.ShapeDtypeStruct((NSL, NPAD, L), jnp.float32),
    mesh=_mesh,
    compiler_params=_sc_params,
    scratch_types=[
        pltpu.VMEM_SHARED((NPAD, L), jnp.float32),
        pltpu.VMEM((2, KI, CHUNK), jnp.int32),
        pltpu.VMEM((2, KI, CHUNK), jnp.int32),
        pltpu.VMEM((2, KI, CHUNK, L), jnp.float32),
        pltpu.VMEM((WB, L), jnp.float32),
        pltpu.SemaphoreType.DMA,
        pltpu.SemaphoreType.DMA,
        pltpu.SemaphoreType.DMA,
        pltpu.SemaphoreType.DMA,
    ],
)


RBLK = 2000


def _proj_body(x_ref, w_ref, b_ref, o_ref):
    o_ref[...] = (jnp.dot(x_ref[...], w_ref[...],
                          preferred_element_type=jnp.float32) + b_ref[...])


_proj_call = pl.pallas_call(
    _proj_body,
    grid=(N // RBLK,),
    in_specs=[
        pl.BlockSpec((RBLK, D), lambda i: (i, 0)),
        pl.BlockSpec((D, H), lambda i: (0, 0)),
        pl.BlockSpec((1, H), lambda i: (0, 0)),
    ],
    out_specs=pl.BlockSpec((RBLK, H), lambda i: (i, 0)),
    out_shape=jax.ShapeDtypeStruct((N, H), jnp.float32),
)


def _combine_act_body(s0_ref, i0_ref, s1_ref, i1_ref, w0_ref, w1_ref, b_ref,
                      o_ref):
    m0 = jnp.concatenate([s0_ref[k] for k in range(NSL)], -1) * i0_ref[...]
    m1 = jnp.concatenate([s1_ref[k] for k in range(NSL)], -1) * i1_ref[...]
    h = (jnp.dot(m0, w0_ref[...], preferred_element_type=jnp.float32)
         + jnp.dot(m1, w1_ref[...], preferred_element_type=jnp.float32)
         + b_ref[...])
    o_ref[...] = jnp.where(h >= 0.0, h, 0.01 * h)


_combine_act_call = pl.pallas_call(
    _combine_act_body,
    grid=(N // RBLK,),
    in_specs=[
        pl.BlockSpec((NSL, RBLK, L), lambda i: (0, i, 0)),
        pl.BlockSpec((RBLK, 1), lambda i: (i, 0)),
        pl.BlockSpec((NSL, RBLK, L), lambda i: (0, i, 0)),
        pl.BlockSpec((RBLK, 1), lambda i: (i, 0)),
        pl.BlockSpec((H, H), lambda i: (0, 0)),
        pl.BlockSpec((H, H), lambda i: (0, 0)),
        pl.BlockSpec((1, H), lambda i: (0, 0)),
    ],
    out_specs=pl.BlockSpec((RBLK, H), lambda i: (i, 0)),
    out_shape=jax.ShapeDtypeStruct((N, H), jnp.float32),
)


def _combine_head_body(s0_ref, i0_ref, s1_ref, i1_ref, w0_ref, w1_ref, b_ref,
                       wl_ref, bl_ref, o_ref):
    m0 = jnp.concatenate([s0_ref[k] for k in range(NSL)], -1) * i0_ref[...]
    m1 = jnp.concatenate([s1_ref[k] for k in range(NSL)], -1) * i1_ref[...]
    h = (jnp.dot(m0, w0_ref[...], preferred_element_type=jnp.float32)
         + jnp.dot(m1, w1_ref[...], preferred_element_type=jnp.float32)
         + b_ref[...])
    o_ref[...] = (jnp.dot(h, wl_ref[...], preferred_element_type=jnp.float32)
                  + bl_ref[...])


_combine_head_call = pl.pallas_call(
    _combine_head_body,
    grid=(N // RBLK,),
    in_specs=[
        pl.BlockSpec((NSL, RBLK, L), lambda i: (0, i, 0)),
        pl.BlockSpec((RBLK, 1), lambda i: (i, 0)),
        pl.BlockSpec((NSL, RBLK, L), lambda i: (0, i, 0)),
        pl.BlockSpec((RBLK, 1), lambda i: (i, 0)),
        pl.BlockSpec((H, H), lambda i: (0, 0)),
        pl.BlockSpec((H, H), lambda i: (0, 0)),
        pl.BlockSpec((1, H), lambda i: (0, 0)),
        pl.BlockSpec((H, O), lambda i: (0, 0)),
        pl.BlockSpec((1, O), lambda i: (0, 0)),
    ],
    out_specs=pl.BlockSpec((RBLK, O), lambda i: (i, 0)),
    out_shape=jax.ShapeDtypeStruct((N, O), jnp.float32),
)


def _pad_edges(ei):
    pad = EPAD - E
    src = jnp.concatenate([ei[0], jnp.zeros((pad,), jnp.int32)])
    dst = jnp.concatenate([ei[1], jnp.full((pad,), N, jnp.int32)])
    offs = (jnp.arange(NSL, dtype=jnp.int32) * N)[:, None]
    src3d = (src[None, :] + offs).reshape(NSL, EROWS, CHUNK)
    dst2d = dst.reshape(EROWS, CHUNK)
    return src3d, dst2d


def _slice_major(h):
    return h.reshape(N, NSL, L).transpose(1, 0, 2).reshape(NSL * N, L)


def _unslice(s3):
    return s3[:, :N, :].transpose(1, 0, 2).reshape(N, H)


def kernel(x_node, edge_index_e0, edge_index_e1, Wp, bp, W0_e0, b0_e0,
           W0_e1, b0_e1, W1_e0, b1_e0, W1_e1, b1_e1, Wlin, blin):
    src3d_0, dst2d_0 = _pad_edges(edge_index_e0)
    src3d_1, dst2d_1 = _pad_edges(edge_index_e1)

    inv0_w, inv1_w = _count_call(dst2d_0, dst2d_1)
    inv0 = inv0_w[:N, :1]
    inv1 = inv1_w[:N, :1]

    h = _proj_call(x_node, Wp.T, bp[None, :])

    ht = _slice_major(h)
    s00 = _segsum_call(ht, src3d_0, dst2d_0)
    s01 = _segsum_call(ht, src3d_1, dst2d_1)
    h1 = _combine_act_call(s00, inv0, s01, inv1, W0_e0.T, W0_e1.T,
                           (b0_e0 + b0_e1)[None, :])

    h1t = _slice_major(h1)
    s10 = _segsum_call(h1t, src3d_0, dst2d_0)
    s11 = _segsum_call(h1t, src3d_1, dst2d_1)
    out = _combine_head_call(s10, inv0, s11, inv1, W1_e0.T, W1_e1.T,
                             (b1_e0 + b1_e1)[None, :], Wlin.T, blin[None, :])
    return out
